# baseline (speedup 1.0000x reference)
import dataclasses
import functools

import jax
import jax.numpy as jnp
from jax import lax
from jax.experimental import pallas as pl
from jax.experimental.pallas import tpu as pltpu
from jax.experimental.pallas import tpu_sc as plsc

NUM_CORES = 2
NUM_SUBCORES = 16
NUM_WORKERS = NUM_CORES * NUM_SUBCORES
LANES = 16
IDX_ROW = 128
NBUF = 4


def _compiler_params():
    cp = pltpu.CompilerParams()
    if "needs_layout_passes" in pltpu.CompilerParams.__dataclass_fields__:
        cp = dataclasses.replace(cp, needs_layout_passes=False)
    if "use_tc_tiling_on_sc" in pltpu.CompilerParams.__dataclass_fields__:
        cp = dataclasses.replace(cp, use_tc_tiling_on_sc=False)
    return cp


def kernel(item_ids, context_ids, user_emb, item_emb, user_bias, item_bias):
    batch = item_ids.shape[0]
    dim = user_emb.shape[1]
    n = user_emb.shape[0]
    bpw = batch // NUM_WORKERS
    n_chunks = bpw // IDX_ROW
    tiles_i = -(-n // 128)
    n_tiles = (dim // 8) * tiles_i
    tpw = -(-n_tiles // NUM_WORKERS)
    pad_n = tiles_i * 128

    ii = item_ids.astype(jnp.int32)
    ci = context_ids.astype(jnp.int32)
    uT = user_emb.T
    iT = item_emb.T
    ub = user_bias.reshape(-1)
    ib = item_bias.reshape(-1)

    mesh = plsc.VectorSubcoreMesh(core_axis_name="c", subcore_axis_name="s")
    cp = _compiler_params()

    detile_out = jax.ShapeDtypeStruct((n_tiles * 8, 128), jnp.float32)

    @functools.partial(
        pl.kernel,
        out_type=(detile_out, detile_out),
        mesh=mesh,
        compiler_params=cp,
        scratch_types=(
            [pltpu.VMEM((8, 128), jnp.float32) for _ in range(2 * NBUF)]
            + [pltpu.SemaphoreType.DMA] * (2 * NBUF)
        ),
    )
    def detile(uT_hbm, iT_hbm, du_hbm, di_hbm, *scr):
        bufs, sems = scr[:2 * NBUF], scr[2 * NBUF:]
        wid = lax.axis_index("s") * NUM_CORES + lax.axis_index("c")
        g = wid >> 2
        q = wid & 3
        t0 = q * tpw

        def rd(src, b, t):
            @pl.when(t < tiles_i)
            def _():
                pltpu.async_copy(
                    src.at[pl.ds(g * 8, 8), pl.ds(t * 128, 128)],
                    bufs[b], sems[b])

        def wait_rd(dst, b, t):
            @pl.when(t < tiles_i)
            def _():
                pltpu.make_async_copy(
                    dst.at[pl.ds(0, 8), pl.ds(0, 128)], bufs[b],
                    sems[b]).wait()

        def wr(dst, b, t):
            @pl.when(t < tiles_i)
            def _():
                pltpu.async_copy(
                    bufs[b], dst.at[pl.ds((g * tiles_i + t) * 8, 8)], sems[b])

        def wait_wr(dst, b, t):
            @pl.when(t < tiles_i)
            def _():
                pltpu.make_async_copy(
                    bufs[b], dst.at[pl.ds(0, 8)], sems[b]).wait()

        for half, (src, dst) in enumerate(((uT_hbm, du_hbm), (iT_hbm, di_hbm))):
            boff = half * NBUF
            for b in range(NBUF):
                rd(src, boff + b, t0 + b)

            @pl.loop(0, tpw, step=NBUF)
            def _(k):
                for b in range(NBUF):
                    wait_rd(dst, boff + b, t0 + k + b)
                    wr(dst, boff + b, t0 + k + b)
                for b in range(NBUF):
                    wait_wr(dst, boff + b, t0 + k + b)
                    rd(src, boff + b, t0 + k + NBUF + b)

            for b in range(NBUF):
                wait_rd(dst, boff + b,
                        t0 + ((tpw + NBUF - 1) // NBUF) * NBUF + b)

    du_raw, di_raw = detile(uT, iT)
    du = du_raw.reshape(-1)
    di = di_raw.reshape(-1)

    @functools.partial(
        pl.kernel,
        out_type=jax.ShapeDtypeStruct((batch,), jnp.float32),
        mesh=mesh,
        compiler_params=cp,
        scratch_types=[
            pltpu.VMEM((bpw,), jnp.int32),
            pltpu.VMEM((bpw,), jnp.int32),
            pltpu.VMEM((bpw * dim // 2,), jnp.int32),
            pltpu.VMEM((bpw * dim // 2,), jnp.int32),
            pltpu.VMEM((bpw * dim,), jnp.float32),
            pltpu.VMEM((bpw * dim,), jnp.float32),
            pltpu.VMEM((bpw,), jnp.float32),
            pltpu.VMEM((bpw,), jnp.float32),
            pltpu.VMEM((bpw,), jnp.float32),
            pltpu.VMEM((LANES * LANES,), jnp.float32),
            pltpu.SemaphoreType.DMA,
            pltpu.SemaphoreType.DMA,
        ],
    )
    def gather_dot(ii_hbm, ci_hbm, du_hbm, di_hbm, ub_hbm, ib_hbm, out_hbm,
                   idx_u, idx_i, pidx_u, pidx_i, rows_u, rows_i,
                   bias_u, bias_i, out_v, tr_buf, isem, sem):
        wid = lax.axis_index("s") * NUM_CORES + lax.axis_index("c")
        base = wid * bpw

        pltpu.async_copy(ii_hbm.at[pl.ds(base, bpw)], idx_u, isem).wait()
        pltpu.async_copy(ci_hbm.at[pl.ds(base, bpw)], idx_i, isem).wait()

        copies = []
        for j in range(n_chunks):
            copies.append(pltpu.async_copy(
                ub_hbm.at[idx_u.at[pl.ds(j * IDX_ROW, IDX_ROW)]],
                bias_u.at[pl.ds(j * IDX_ROW, IDX_ROW)], isem))
            copies.append(pltpu.async_copy(
                ib_hbm.at[idx_i.at[pl.ds(j * IDX_ROW, IDX_ROW)]],
                bias_i.at[pl.ds(j * IDX_ROW, IDX_ROW)], isem))

        lane_iota = lax.iota(jnp.int32, LANES)
        pats = []
        for g in range(dim // LANES):
            cvec = lane_iota + g * LANES
            pats.append((cvec >> 3) * (tiles_i * 1024) + (cvec & 7) * 128)

        half_rows = bpw // 2
        for ch in (0, half_rows):
            @pl.loop(0, half_rows, step=LANES)
            def _(q):
                ivec_u = idx_u[pl.ds(ch + q, LANES)]
                ivec_i = idx_i[pl.ds(ch + q, LANES)]
                for r16 in range(LANES):
                    iu = ivec_u[r16]
                    iv = ivec_i[r16]
                    bu = ((iu >> 7) << 10) + (iu & 127)
                    bv = ((iv >> 7) << 10) + (iv & 127)
                    for g in range(dim // LANES):
                        o = pl.ds((q + r16) * dim + g * LANES, LANES)
                        pidx_u[o] = pats[g] + bu
                        pidx_i[o] = pats[g] + bv

            @pl.loop(0, half_rows * dim, step=IDX_ROW)
            def _(o):
                pltpu.async_copy(
                    du_hbm.at[pidx_u.at[pl.ds(o, IDX_ROW)]],
                    rows_u.at[pl.ds(ch * dim + o, IDX_ROW)], sem)
                pltpu.async_copy(
                    di_hbm.at[pidx_i.at[pl.ds(o, IDX_ROW)]],
                    rows_i.at[pl.ds(ch * dim + o, IDX_ROW)], sem)

            @pl.loop(0, half_rows * dim, step=IDX_ROW)
            def _(o):
                pltpu.make_async_copy(
                    du_hbm.at[pl.ds(0, IDX_ROW)],
                    rows_u.at[pl.ds(ch * dim + o, IDX_ROW)], sem).wait()
                pltpu.make_async_copy(
                    di_hbm.at[pl.ds(0, IDX_ROW)],
                    rows_i.at[pl.ds(ch * dim + o, IDX_ROW)], sem).wait()

        for c in copies:
            c.wait()

        @pl.loop(0, bpw, step=LANES)
        def _(blk):
            for r16 in range(LANES):
                acc = (rows_u[pl.ds((blk + r16) * dim, LANES)]
                       * rows_i[pl.ds((blk + r16) * dim, LANES)])
                for c in range(LANES, dim, LANES):
                    acc = acc + (rows_u[pl.ds((blk + r16) * dim + c, LANES)]
                                 * rows_i[pl.ds((blk + r16) * dim + c, LANES)])
                plsc.store_scatter(tr_buf, [lane_iota * LANES + r16], acc)
            s = bias_u[pl.ds(blk, LANES)] + bias_i[pl.ds(blk, LANES)]
            for l in range(LANES):
                s = s + tr_buf[pl.ds(l * LANES, LANES)]
            out_v[pl.ds(blk, LANES)] = s

        pltpu.sync_copy(out_v, out_hbm.at[pl.ds(base, bpw)])

    return gather_dot(ii, ci, du, di, ub, ib)


# speedup vs baseline: 9.9371x; 9.9371x over previous
import dataclasses
import functools

import jax
import jax.numpy as jnp
from jax import lax
from jax.experimental import pallas as pl
from jax.experimental.pallas import tpu as pltpu
from jax.experimental.pallas import tpu_sc as plsc

NUM_CORES = 2
NUM_SUBCORES = 16
NUM_WORKERS = NUM_CORES * NUM_SUBCORES
LANES = 16
IDX_ROW = 128


def kernel(item_ids, context_ids, user_emb, item_emb, user_bias, item_bias):
    batch = item_ids.shape[0]
    dim = user_emb.shape[1]
    bpw = batch // NUM_WORKERS
    n_chunks = bpw // IDX_ROW

    ii = item_ids.astype(jnp.int32).reshape(NUM_WORKERS * n_chunks, IDX_ROW)
    ci = context_ids.astype(jnp.int32).reshape(NUM_WORKERS * n_chunks, IDX_ROW)
    ub = user_bias.reshape(-1)
    ib = item_bias.reshape(-1)

    mesh = plsc.VectorSubcoreMesh(core_axis_name="c", subcore_axis_name="s")
    cp = pltpu.CompilerParams()
    if "needs_layout_passes" in pltpu.CompilerParams.__dataclass_fields__:
        cp = dataclasses.replace(cp, needs_layout_passes=False)
    if "use_tc_tiling_on_sc" in pltpu.CompilerParams.__dataclass_fields__:
        cp = dataclasses.replace(cp, use_tc_tiling_on_sc=False)

    @functools.partial(
        pl.kernel,
        out_type=jax.ShapeDtypeStruct((batch,), jnp.float32),
        mesh=mesh,
        compiler_params=cp,
        scratch_types=[
            pltpu.VMEM((n_chunks, IDX_ROW), jnp.int32),
            pltpu.VMEM((n_chunks, IDX_ROW), jnp.int32),
            pltpu.VMEM((bpw, dim), jnp.float32),
            pltpu.VMEM((bpw, dim), jnp.float32),
            pltpu.VMEM((bpw,), jnp.float32),
            pltpu.VMEM((bpw,), jnp.float32),
            pltpu.VMEM((bpw,), jnp.float32),
            pltpu.VMEM((LANES * LANES,), jnp.float32),
            pltpu.SemaphoreType.DMA,
        ],
    )
    def glove_kernel(ii_hbm, ci_hbm, ue_hbm, ie_hbm, ub_hbm, ib_hbm, out_hbm,
                     idx_u, idx_i, rows_u, rows_i, bias_u, bias_i, out_v,
                     tr_buf, sem):
        wid = lax.axis_index("s") * NUM_CORES + lax.axis_index("c")
        base = wid * bpw

        pltpu.sync_copy(ii_hbm.at[pl.ds(wid * n_chunks, n_chunks)], idx_u)
        pltpu.sync_copy(ci_hbm.at[pl.ds(wid * n_chunks, n_chunks)], idx_i)

        copies = []
        for j in range(n_chunks):
            copies.append(pltpu.async_copy(
                ue_hbm.at[idx_u.at[j]], rows_u.at[pl.ds(j * IDX_ROW, IDX_ROW)], sem))
            copies.append(pltpu.async_copy(
                ie_hbm.at[idx_i.at[j]], rows_i.at[pl.ds(j * IDX_ROW, IDX_ROW)], sem))
            copies.append(pltpu.async_copy(
                ub_hbm.at[idx_u.at[j]], bias_u.at[pl.ds(j * IDX_ROW, IDX_ROW)], sem))
            copies.append(pltpu.async_copy(
                ib_hbm.at[idx_i.at[j]], bias_i.at[pl.ds(j * IDX_ROW, IDX_ROW)], sem))
        for c in copies:
            c.wait()

        lane_iota = lax.iota(jnp.int32, LANES)

        @pl.loop(0, bpw, step=LANES)
        def _(blk):
            for r16 in range(LANES):
                acc = (rows_u[blk + r16, pl.ds(0, LANES)]
                       * rows_i[blk + r16, pl.ds(0, LANES)])
                for c in range(LANES, dim, LANES):
                    acc = acc + (rows_u[blk + r16, pl.ds(c, LANES)]
                                 * rows_i[blk + r16, pl.ds(c, LANES)])
                plsc.store_scatter(tr_buf, [lane_iota * LANES + r16], acc)
            s = bias_u[pl.ds(blk, LANES)] + bias_i[pl.ds(blk, LANES)]
            for l in range(LANES):
                s = s + tr_buf[pl.ds(l * LANES, LANES)]
            out_v[pl.ds(blk, LANES)] = s

        pltpu.sync_copy(out_v, out_hbm.at[pl.ds(base, bpw)])

    return glove_kernel(ii, ci, user_emb, item_emb, ub, ib)
